# baseline (speedup 1.0000x reference)
import functools

import jax
import jax.numpy as jnp
from jax import lax
from jax.experimental import pallas as pl
from jax.experimental.pallas import tpu as pltpu

_NCH = 4


def _probe_kernel(*refs, seq_len, chunk):
    x_refs = refs[:_NCH]
    xnorm_ref = refs[_NCH]
    acc = None
    for c, xr in enumerate(x_refs):
        x = xr[...]
        if (c + 1) * chunk > seq_len:
            pos = c * chunk + lax.broadcasted_iota(jnp.int32, x.shape, 1)
            x = jnp.where(pos < seq_len, x, jnp.float32(0.0))
        s = jnp.sum(x, axis=1)
        acc = s if acc is None else acc + s
    x_mean = acc * jnp.float32(1.0 / seq_len)
    x_sq = jnp.sum(x_mean * x_mean, axis=-1, keepdims=True)
    xnorm_ref[...] = x_mean * lax.rsqrt(jnp.maximum(x_sq, jnp.float32(1e-12)))


def kernel(x_embed, prompt, prompt_key):
    B, S, D = x_embed.shape
    TB = 16
    NB = B // TB
    chunk = 56
    xnorm = pl.pallas_call(
        functools.partial(_probe_kernel, seq_len=S, chunk=chunk),
        out_shape=jax.ShapeDtypeStruct((B, D), jnp.float32),
        grid=(NB,),
        in_specs=[
            pl.BlockSpec((TB, chunk, D),
                         functools.partial(lambda c, i: (i, c, 0), c))
            for c in range(_NCH)
        ],
        out_specs=pl.BlockSpec((TB, D), lambda i: (i, 0)),
        compiler_params=pltpu.CompilerParams(
            dimension_semantics=("parallel",),
            vmem_limit_bytes=int(64 * 1024 * 1024 * 0.9)),
    )(*([x_embed] * _NCH))
    return {'x_embed_norm': xnorm}
